# baseline (speedup 1.0000x reference)
import functools
import math

import jax
import jax.numpy as jnp
from jax.experimental import pallas as pl
from jax.experimental.pallas import tpu as pltpu

D_MODEL = 1024
N_IDX = 4 * 8192
BASE = 10000.0
TC_BLK = 256


def _tc_body(t_ref, freq_ref, phase_ref, out_ref):
    tv = t_ref[0, 0, :].astype(jnp.float32)
    f = freq_ref[0, :]
    ph = phase_ref[0, :]
    ang = tv[:, None] * f[None, :] + ph[None, :]
    out_ref[...] = jnp.sin(ang)


def _tc_compute(t_flat):
    n = t_flat.shape[0]
    col = jnp.arange(D_MODEL, dtype=jnp.float32)
    fexp = jnp.floor(col / 2.0) * 2.0
    freq = jnp.exp(fexp * (-math.log(BASE) / D_MODEL)).reshape(1, D_MODEL)
    phase = (jnp.arange(D_MODEL) % 2).astype(jnp.float32).reshape(1, D_MODEL) * (
        math.pi / 2.0)
    t3 = t_flat.reshape(n // TC_BLK, 1, TC_BLK)
    return pl.pallas_call(
        _tc_body,
        grid=(n // TC_BLK,),
        in_specs=[
            pl.BlockSpec((1, 1, TC_BLK), lambda i: (i, 0, 0)),
            pl.BlockSpec((1, D_MODEL), lambda i: (0, 0)),
            pl.BlockSpec((1, D_MODEL), lambda i: (0, 0)),
        ],
        out_specs=pl.BlockSpec((TC_BLK, D_MODEL), lambda i: (i, 0)),
        out_shape=jax.ShapeDtypeStruct((n, D_MODEL), jnp.float32),
    )(t3, freq, phase)


@jax.jit
def kernel(t, pe):
    t_flat = t.reshape(-1)
    out = _tc_compute(t_flat)
    return out.reshape(t.shape + (D_MODEL,))


# speedup vs baseline: 4.4228x; 4.4228x over previous
import functools
import math

import jax
import jax.numpy as jnp
from jax.experimental import pallas as pl
from jax.experimental.pallas import tpu as pltpu

D_MODEL = 1024
N_IDX = 4 * 8192
BASE = 10000.0
TC_BLK = 256


def _tc_body(t_ref, freq_ref, phase_ref, out_ref):
    tv = t_ref[0, 0, :].astype(jnp.float32)
    f = freq_ref[0, :]
    ph = phase_ref[0, :]
    ang = tv[:, None] * f[None, :] + ph[None, :]
    u = ang * (1.0 / (2.0 * math.pi))
    r = u - jnp.round(u)
    p = 16.0 * r * (0.5 - jnp.abs(r))
    out_ref[...] = p * (0.775 + 0.225 * jnp.abs(p))


def _tc_compute(t_flat):
    n = t_flat.shape[0]
    col = jnp.arange(D_MODEL, dtype=jnp.float32)
    fexp = jnp.floor(col / 2.0) * 2.0
    freq = jnp.exp(fexp * (-math.log(BASE) / D_MODEL)).reshape(1, D_MODEL)
    phase = (jnp.arange(D_MODEL) % 2).astype(jnp.float32).reshape(1, D_MODEL) * (
        math.pi / 2.0)
    t3 = t_flat.reshape(n // TC_BLK, 1, TC_BLK)
    return pl.pallas_call(
        _tc_body,
        grid=(n // TC_BLK,),
        in_specs=[
            pl.BlockSpec((1, 1, TC_BLK), lambda i: (i, 0, 0)),
            pl.BlockSpec((1, D_MODEL), lambda i: (0, 0)),
            pl.BlockSpec((1, D_MODEL), lambda i: (0, 0)),
        ],
        out_specs=pl.BlockSpec((TC_BLK, D_MODEL), lambda i: (i, 0)),
        out_shape=jax.ShapeDtypeStruct((n, D_MODEL), jnp.float32),
    )(t3, freq, phase)


@jax.jit
def kernel(t, pe):
    t_flat = t.reshape(-1)
    out = _tc_compute(t_flat)
    return out.reshape(t.shape + (D_MODEL,))
